# baseline (speedup 1.0000x reference)
import jax
import jax.numpy as jnp
from jax import lax
from jax.experimental import pallas as pl
from jax.experimental.pallas import tpu as pltpu
from jax.experimental.pallas import tpu_sc as plsc

P = 1024
D = 256
N_NODES = 511
N_PAD = 512
DEPTH = 8
NW = 32
PPW = P // NW
L = 16


def _scores_body(x_ref, nodes_ref, out_ref):
    x = x_ref[...]
    xp = x.reshape(32, 16, 32, 16).transpose(0, 2, 1, 3).reshape(P, D)
    n = nodes_ref[...]
    nt = jnp.concatenate(
        [n, jnp.zeros((N_PAD - N_NODES, D), jnp.float32)], axis=0).T
    dots = lax.dot_general(
        xp, nt, (((1,), (0,)), ((), ())),
        preferred_element_type=jnp.float32,
        precision=lax.Precision.HIGHEST,
    )
    norms = jnp.sum(nt * nt, axis=0)
    out_ref[...] = norms[None, :] - 2.0 * dots


def _walk_body(s_hbm, out_hbm, s_v, res_v, sem_lo, sem_hi):
    c = lax.axis_index("c")
    s = lax.axis_index("s")
    wid = s * 2 + c
    base = wid * PPW
    cp_lo = pltpu.make_async_copy(
        s_hbm.at[pl.ds(base, PPW), pl.ds(0, 256)],
        s_v.at[:, pl.ds(0, 256)], sem_lo)
    cp_hi = pltpu.make_async_copy(
        s_hbm.at[pl.ds(base, PPW), pl.ds(256, 256)],
        s_v.at[:, pl.ds(256, 256)], sem_hi)
    cp_lo.start()
    cp_hi.start()
    cp_lo.wait()
    curs = []
    for g in range(PPW // L):
        rows = lax.iota(jnp.int32, L) + g * L
        cur = jnp.zeros((L,), jnp.int32)
        for _ in range(DEPTH - 1):
            c1 = 2 * cur + 1
            c2 = c1 + 1
            s1 = plsc.load_gather(s_v, [rows, c1])
            s2 = plsc.load_gather(s_v, [rows, c2])
            cur = jnp.where(s1 <= s2, c1, c2)
        curs.append((rows, cur))
    cp_hi.wait()
    for g, (rows, cur) in enumerate(curs):
        c1 = 2 * cur + 1
        c2 = c1 + 1
        s1 = plsc.load_gather(s_v, [rows, c1])
        s2 = plsc.load_gather(s_v, [rows, c2])
        cur = jnp.where(s1 <= s2, c1, c2)
        res_v[pl.ds(g * L, L)] = cur
    pltpu.sync_copy(res_v, out_hbm.at[pl.ds(base, PPW)])


def kernel(X, nodes):
    scores = pl.pallas_call(
        _scores_body,
        out_shape=jax.ShapeDtypeStruct((P, N_PAD), jnp.float32),
    )(X, nodes)
    walk = pl.kernel(
        _walk_body,
        out_type=jax.ShapeDtypeStruct((P,), jnp.int32),
        mesh=plsc.VectorSubcoreMesh(core_axis_name="c", subcore_axis_name="s"),
        scratch_types=[
            pltpu.VMEM((PPW, N_PAD), jnp.float32),
            pltpu.VMEM((PPW,), jnp.int32),
            pltpu.SemaphoreType.DMA,
            pltpu.SemaphoreType.DMA,
        ],
        compiler_params=pltpu.CompilerParams(
            needs_layout_passes=False,
            disable_bounds_checks=True,
            skip_device_barrier=True,
        ),
    )
    leaf = walk(scores)
    return leaf.reshape(P, 1)


# speedup vs baseline: 1.0096x; 1.0096x over previous
import jax
import jax.numpy as jnp
from jax import lax
from jax.experimental import pallas as pl
from jax.experimental.pallas import tpu as pltpu
from jax.experimental.pallas import tpu_sc as plsc

P = 1024
D = 256
N_NODES = 511
N_PAD = 512
DEPTH = 8
NW = 32
PPW = P // NW
L = 16


def _scores_body(x_ref, nodes_ref, out_ref):
    n = nodes_ref[...]
    nt = jnp.concatenate(
        [n, jnp.zeros((N_PAD - N_NODES, D), jnp.float32)], axis=0).T
    norms = jnp.sum(nt * nt, axis=0)
    for h in range(4):
        x = x_ref[pl.ds(128 * h, 128), :]
        xp = x.reshape(8, 16, 32, 16).transpose(0, 2, 1, 3).reshape(P // 4, D)
        dots = lax.dot_general(
            xp, nt, (((1,), (0,)), ((), ())),
            preferred_element_type=jnp.float32,
            precision=lax.Precision.HIGHEST,
        )
        out_ref[pl.ds(P // 4 * h, P // 4), :] = norms[None, :] - 2.0 * dots


def _walk_body(s_hbm, out_hbm, s_v, res_v, sem_lo, sem_hi):
    c = lax.axis_index("c")
    s = lax.axis_index("s")
    wid = s * 2 + c
    base = wid * PPW
    cp_lo = pltpu.make_async_copy(
        s_hbm.at[pl.ds(base, PPW), pl.ds(0, 256)],
        s_v.at[:, pl.ds(0, 256)], sem_lo)
    cp_hi = pltpu.make_async_copy(
        s_hbm.at[pl.ds(base, PPW), pl.ds(256, 256)],
        s_v.at[:, pl.ds(256, 256)], sem_hi)
    cp_lo.start()
    cp_hi.start()
    cp_lo.wait()
    curs = []
    for g in range(PPW // L):
        rows = lax.iota(jnp.int32, L) + g * L
        cur = jnp.zeros((L,), jnp.int32)
        for _ in range(DEPTH - 1):
            c1 = 2 * cur + 1
            c2 = c1 + 1
            s1 = plsc.load_gather(s_v, [rows, c1])
            s2 = plsc.load_gather(s_v, [rows, c2])
            cur = jnp.where(s1 <= s2, c1, c2)
        curs.append((rows, cur))
    cp_hi.wait()
    for g, (rows, cur) in enumerate(curs):
        c1 = 2 * cur + 1
        c2 = c1 + 1
        s1 = plsc.load_gather(s_v, [rows, c1])
        s2 = plsc.load_gather(s_v, [rows, c2])
        cur = jnp.where(s1 <= s2, c1, c2)
        res_v[pl.ds(g * L, L)] = cur
    pltpu.sync_copy(res_v, out_hbm.at[pl.ds(base, PPW)])


def kernel(X, nodes):
    scores = pl.pallas_call(
        _scores_body,
        out_shape=jax.ShapeDtypeStruct((P, N_PAD), jnp.float32),
    )(X, nodes)
    walk = pl.kernel(
        _walk_body,
        out_type=jax.ShapeDtypeStruct((P,), jnp.int32),
        mesh=plsc.VectorSubcoreMesh(core_axis_name="c", subcore_axis_name="s"),
        scratch_types=[
            pltpu.VMEM((PPW, N_PAD), jnp.float32),
            pltpu.VMEM((PPW,), jnp.int32),
            pltpu.SemaphoreType.DMA,
            pltpu.SemaphoreType.DMA,
        ],
        compiler_params=pltpu.CompilerParams(needs_layout_passes=False),
    )
    leaf = walk(scores)
    return leaf.reshape(P, 1)
